# baseline (speedup 1.0000x reference)
import functools

import jax
import jax.numpy as jnp
from jax import lax
from jax.experimental import pallas as pl
from jax.experimental.pallas import tpu as pltpu
from jax.experimental.pallas import tpu_sc as plsc

HIDDEN = 2048
VOCAB = 3
NUM_CORES = 2
NUM_SUBCORES = 16
NW = NUM_CORES * NUM_SUBCORES
BLK = 16
LAG = 4


@functools.lru_cache(maxsize=None)
def _make(total: int):
    per_w = total // NW
    n_blk = per_w // BLK
    assert total % NW == 0 and per_w % BLK == 0 and n_blk > LAG
    mesh = plsc.VectorSubcoreMesh(
        core_axis_name="c",
        subcore_axis_name="s",
        num_cores=NUM_CORES,
        num_subcores=NUM_SUBCORES,
    )

    @functools.partial(
        pl.kernel,
        out_type=jax.ShapeDtypeStruct((total, HIDDEN), jnp.float32),
        mesh=mesh,
        scratch_types=[
            pltpu.VMEM((VOCAB * VOCAB * 2, HIDDEN), jnp.float32),
            pltpu.VMEM((per_w,), jnp.int32),
            pltpu.VMEM((BLK, HIDDEN), jnp.float32),
            pltpu.SemaphoreType.DMA,
        ],
    )
    def k(y_hbm, table_hbm, out_hbm, pair_v, idx_v, dummy_v, sem):
        cid = lax.axis_index("c")
        sid = lax.axis_index("s")
        wid = sid * NUM_CORES + cid
        base = wid * per_w

        for a in range(VOCAB):
            for b in range(VOCAB):
                p = VOCAB * a + b
                pltpu.sync_copy(table_hbm.at[pl.ds(a, 1)],
                                pair_v.at[pl.ds(2 * p, 1)])
                pltpu.sync_copy(table_hbm.at[pl.ds(b, 1)],
                                pair_v.at[pl.ds(2 * p + 1, 1)])
        pltpu.sync_copy(y_hbm.at[pl.ds(base, per_w)], idx_v)

        def block(blk, carry):
            v = idx_v[pl.ds(blk * BLK, BLK)]
            tok0 = base + blk * BLK
            for l in range(BLK // 2):
                p = VOCAB * v[2 * l] + v[2 * l + 1]
                pltpu.async_copy(
                    pair_v.at[pl.ds(2 * p, 2)],
                    out_hbm.at[pl.ds(tok0 + 2 * l, 2)],
                    sem,
                )

            @pl.when(blk >= LAG)
            def _():
                d0 = base + (blk - LAG) * BLK
                pltpu.make_async_copy(
                    dummy_v, out_hbm.at[pl.ds(d0, BLK)], sem
                ).wait()

            return carry

        lax.fori_loop(0, n_blk, block, 0)
        for t in range(LAG):
            d0 = base + (n_blk - LAG + t) * BLK
            pltpu.make_async_copy(
                dummy_v, out_hbm.at[pl.ds(d0, BLK)], sem
            ).wait()

    return k


def kernel(y, table):
    B, S = y.shape
    total = B * S
    yf = y.reshape(total).astype(jnp.int32)
    out = _make(total)(yf, table)
    return out.reshape(B, S, HIDDEN)


# speedup vs baseline: 1.1877x; 1.1877x over previous
import functools

import jax
import jax.numpy as jnp
from jax import lax
from jax.experimental import pallas as pl
from jax.experimental.pallas import tpu as pltpu
from jax.experimental.pallas import tpu_sc as plsc

HIDDEN = 2048
VOCAB = 3
NUM_CORES = 2
NUM_SUBCORES = 16
NW = NUM_CORES * NUM_SUBCORES
BLK = 16
LAG = 8


@functools.lru_cache(maxsize=None)
def _make(total: int):
    per_w = total // NW
    n_blk = per_w // BLK
    assert total % NW == 0 and per_w % BLK == 0 and n_blk > LAG
    mesh = plsc.VectorSubcoreMesh(
        core_axis_name="c",
        subcore_axis_name="s",
        num_cores=NUM_CORES,
        num_subcores=NUM_SUBCORES,
    )

    @functools.partial(
        pl.kernel,
        out_type=jax.ShapeDtypeStruct((total, HIDDEN), jnp.float32),
        mesh=mesh,
        scratch_types=[
            pltpu.VMEM((VOCAB, HIDDEN), jnp.float32),
            pltpu.VMEM((per_w,), jnp.int32),
            pltpu.VMEM((BLK, HIDDEN), jnp.float32),
            pltpu.SemaphoreType.DMA,
        ],
    )
    def k(y_hbm, table_hbm, out_hbm, table_v, idx_v, dummy_v, sem):
        cid = lax.axis_index("c")
        sid = lax.axis_index("s")
        wid = sid * NUM_CORES + cid
        base = wid * per_w

        pltpu.sync_copy(table_hbm, table_v)
        pltpu.sync_copy(y_hbm.at[pl.ds(base, per_w)], idx_v)

        def block(blk, carry):
            v = idx_v[pl.ds(blk * BLK, BLK)]
            tok0 = base + blk * BLK
            for l in range(BLK):
                s = v[l]
                pltpu.async_copy(
                    table_v.at[pl.ds(s, 1)],
                    out_hbm.at[pl.ds(tok0 + l, 1)],
                    sem,
                )

            @pl.when(blk >= LAG)
            def _():
                d0 = base + (blk - LAG) * BLK
                pltpu.make_async_copy(
                    dummy_v, out_hbm.at[pl.ds(d0, BLK)], sem
                ).wait()

            return carry

        lax.fori_loop(0, n_blk, block, 0)
        for t in range(LAG):
            d0 = base + (n_blk - LAG + t) * BLK
            pltpu.make_async_copy(
                dummy_v, out_hbm.at[pl.ds(d0, BLK)], sem
            ).wait()

    return k


def kernel(y, table):
    B, S = y.shape
    total = B * S
    yf = y.reshape(total).astype(jnp.int32)
    out = _make(total)(yf, table)
    return out.reshape(B, S, HIDDEN)
